# baseline (speedup 1.0000x reference)
import functools

import jax
import jax.numpy as jnp
from jax import lax
from jax.experimental import pallas as pl
from jax.experimental.pallas import tpu as pltpu
from jax.experimental.pallas import tpu_sc as plsc

N_NODES = 10000
N_PAD = 10240
N_EDGES = 320000
D = 128
HD = D // 2

NC = 2
NS = 16
NW = NC * NS
CH = 80
CH2 = CH // 2
ROWS_PER_TILE = N_PAD // NS

NB = 5
EB = N_EDGES // NB


def _pipelined_chunks(nch, start, wait0, wait1, finish):
    start(0, 0)
    npair = (nch - 1) // 2 if nch % 2 else nch // 2 - 1

    def pair(jj, carry):
        j0 = 2 * jj
        start(j0 + 1, 1)
        wait0()
        finish(j0, 0)
        start(j0 + 2, 0)
        wait1()
        finish(j0 + 1, 1)
        return carry

    lax.fori_loop(0, npair, pair, 0)
    if nch % 2:
        wait0()
        finish(nch - 1, 0)
    else:
        j = 2 * npair
        start(j + 1, 1)
        wait0()
        finish(j, 0)
        wait1()
        finish(j + 1, 1)


def _pack_i32(v32):
    b = jax.lax.bitcast_convert_type(v32, jnp.int32) + jnp.int32(32768)
    lo = jax.lax.shift_right_logical(b[:, :HD], 16)
    hi = b[:, HD:] & jnp.int32(-65536)
    return lo | hi


def _unpack_f32(w):
    lo = jax.lax.bitcast_convert_type(jax.lax.shift_left(w, 16), jnp.float32)
    hi = jax.lax.bitcast_convert_type(w & jnp.int32(-65536), jnp.float32)
    return lo, hi


def _pre_body(x_ref, w1a_ref, w1b_ref, b1_ref, xa_ref, xb_ref):
    x = x_ref[...]
    xa_ref[...] = _pack_i32(
        jnp.dot(x, w1a_ref[...], preferred_element_type=jnp.float32)
        + b1_ref[...]
    )
    xb_ref[...] = _pack_i32(
        jnp.dot(x, w1b_ref[...], preferred_element_type=jnp.float32)
    )


def _precompute(x, w1a, w1b, b1):
    return pl.pallas_call(
        _pre_body,
        out_shape=[
            jax.ShapeDtypeStruct((N_NODES, HD), jnp.int32),
            jax.ShapeDtypeStruct((N_NODES, HD), jnp.int32),
        ],
    )(x, w1a, w1b, b1)


def _gather_body(ept, nch, row3d, col3d, xa, xb, g,
                 ir, ic, ba0, bb0, ba1, bb1, go0, go1,
                 sa0, sb0, sa1, sb1):
    wid = lax.axis_index("s") * NC + lax.axis_index("c")
    base = wid * ept
    pltpu.sync_copy(row3d.at[wid], ir)
    pltpu.sync_copy(col3d.at[wid], ic)
    bufs = ((ba0, bb0, go0, sa0, sb0), (ba1, bb1, go1, sa1, sb1))

    def start(j, slot):
        ba, bb, _, sa, sb = bufs[slot]
        pltpu.async_copy(xa.at[ir.at[j]], ba, sa)
        pltpu.async_copy(xb.at[ic.at[j]], bb, sb)

    def wait(slot):
        ba, bb, _, sa, sb = bufs[slot]
        pltpu.make_async_copy(xa.at[ir.at[0]], ba, sa).wait()
        pltpu.make_async_copy(xb.at[ic.at[0]], bb, sb).wait()

    def finish(j, slot):
        ba, bb, go = bufs[slot][:3]

        def addrow(r2, carry):
            for half in range(2):
                r = 2 * r2 + half
                for k in range(HD // 16):
                    sl = pl.ds(k * 16, 16)
                    a = plsc.bitcast(ba[r, sl], jnp.bfloat16)
                    b = plsc.bitcast(bb[r, sl], jnp.bfloat16)
                    go[r2, pl.ds(half * HD + k * 16, 16)] = plsc.bitcast(
                        a + b, jnp.int32
                    )
            return carry

        lax.fori_loop(0, CH2, addrow, 0)
        pltpu.sync_copy(go, g.at[pl.ds((base + j * CH) // 2, CH2)])

    _pipelined_chunks(nch, start, lambda: wait(0), lambda: wait(1), finish)


def _gather(row3d, col3d, xa, xb, n_edges):
    ept = n_edges // NW
    nch = ept // CH
    mesh = plsc.VectorSubcoreMesh(core_axis_name="c", subcore_axis_name="s")
    f = pl.kernel(
        functools.partial(_gather_body, ept, nch),
        out_type=jax.ShapeDtypeStruct((n_edges // 2, D), jnp.int32),
        mesh=mesh,
        scratch_types=[
            pltpu.VMEM((nch, CH), jnp.int32),
            pltpu.VMEM((nch, CH), jnp.int32),
            pltpu.VMEM((CH, HD), jnp.int32),
            pltpu.VMEM((CH, HD), jnp.int32),
            pltpu.VMEM((CH, HD), jnp.int32),
            pltpu.VMEM((CH, HD), jnp.int32),
            pltpu.VMEM((CH2, D), jnp.int32),
            pltpu.VMEM((CH2, D), jnp.int32),
            pltpu.SemaphoreType.DMA,
            pltpu.SemaphoreType.DMA,
            pltpu.SemaphoreType.DMA,
            pltpu.SemaphoreType.DMA,
        ],
        compiler_params=pltpu.CompilerParams(
            use_tc_tiling_on_sc=False, needs_layout_passes=False
        ),
    )
    return f(row3d, col3d, xa, xb)


BE = 1280
BE2 = BE // 2


def _mlp_body(g_ref, w2_ref, b2_ref, he_ref, ho_ref):
    lo, hi = _unpack_f32(g_ref[...])
    ge = jnp.concatenate([lo[:, :HD], hi[:, :HD]], axis=1)
    go = jnp.concatenate([lo[:, HD:], hi[:, HD:]], axis=1)
    w2 = w2_ref[...]
    b2 = b2_ref[...]
    for gg, out_ref in ((ge, he_ref), (go, ho_ref)):
        a = jax.nn.softplus(gg)
        z = jnp.dot(a, w2, preferred_element_type=jnp.float32) + b2
        out_ref[...] = jax.nn.softplus(z)


def _mlp(g, w2, b2):
    n2 = g.shape[0]
    return pl.pallas_call(
        _mlp_body,
        grid=(n2 // BE2,),
        in_specs=[
            pl.BlockSpec((BE2, D), lambda i: (i, 0)),
            pl.BlockSpec((D, D), lambda i: (0, 0)),
            pl.BlockSpec((1, D), lambda i: (0, 0)),
        ],
        out_specs=[
            pl.BlockSpec((BE2, D), lambda i: (i, 0)),
            pl.BlockSpec((BE2, D), lambda i: (i, 0)),
        ],
        out_shape=[
            jax.ShapeDtypeStruct((n2, D), jnp.float32),
            jax.ShapeDtypeStruct((n2, D), jnp.float32),
        ],
    )(g, w2, b2)


def _scatter_body(ept, nch, ir3_e, ir3_o, he, ho, parts, cnts,
                  ire, iro, he0, ho0, he1, ho1, ones, acc, acc_c,
                  se0, so0, se1, so1):
    c = lax.axis_index("c")
    s = lax.axis_index("s")
    wid = s * NC + c
    base2 = wid * ept // 2

    zv = jnp.zeros((16,), jnp.float32)
    ov = jnp.ones((16,), jnp.float32)

    def fill_z(r, carry):
        for k in range(D // 16):
            he0[r, pl.ds(k * 16, 16)] = zv
        return carry

    lax.fori_loop(0, CH2, fill_z, 0)

    def fill_zc(r, carry):
        ones[r, :] = zv
        return carry

    lax.fori_loop(0, CH2, fill_zc, 0)

    def zero_stripe(t, carry):
        off = s * ROWS_PER_TILE + t * CH2
        pltpu.sync_copy(he0, acc.at[pl.ds(off, CH2)])
        return carry

    lax.fori_loop(0, ROWS_PER_TILE // CH2, zero_stripe, 0)

    def zero_stripe_c(t, carry):
        off = s * ROWS_PER_TILE + t * CH2
        pltpu.sync_copy(ones, acc_c.at[pl.ds(off, CH2)])
        return carry

    lax.fori_loop(0, ROWS_PER_TILE // CH2, zero_stripe_c, 0)

    def fill_ones(r, carry):
        ones[r, :] = ov
        return carry

    lax.fori_loop(0, CH2, fill_ones, 0)
    pltpu.sync_copy(ir3_e.at[wid], ire)
    pltpu.sync_copy(ir3_o.at[wid], iro)
    plsc.subcore_barrier()

    bufs = ((he0, ho0, se0, so0), (he1, ho1, se1, so1))

    def start(j, slot):
        hbe, hbo, se, so = bufs[slot]
        pltpu.async_copy(he.at[pl.ds(base2 + j * CH2, CH2)], hbe, se)
        pltpu.async_copy(ho.at[pl.ds(base2 + j * CH2, CH2)], hbo, so)

    def wait(slot):
        hbe, hbo, se, so = bufs[slot]
        pltpu.make_async_copy(he.at[pl.ds(0, CH2)], hbe, se).wait()
        pltpu.make_async_copy(ho.at[pl.ds(0, CH2)], hbo, so).wait()

    def finish(j, slot):
        hbe, hbo = bufs[slot][:2]
        pltpu.sync_copy(hbe, acc.at[ire.at[j]], add=True)
        pltpu.sync_copy(hbo, acc.at[iro.at[j]], add=True)
        pltpu.sync_copy(ones, acc_c.at[ire.at[j]], add=True)
        pltpu.sync_copy(ones, acc_c.at[iro.at[j]], add=True)

    _pipelined_chunks(nch, start, lambda: wait(0), lambda: wait(1), finish)
    plsc.subcore_barrier()

    stripe = pl.ds(s * ROWS_PER_TILE, ROWS_PER_TILE)
    pltpu.sync_copy(acc.at[stripe], parts.at[c].at[stripe])
    pltpu.sync_copy(acc_c.at[stripe], cnts.at[c].at[stripe])


def _scatter(ir3_e, ir3_o, he, ho):
    n_edges = 2 * he.shape[0]
    ept = n_edges // NW
    nch = ept // CH
    mesh = plsc.VectorSubcoreMesh(core_axis_name="c", subcore_axis_name="s")
    f = pl.kernel(
        functools.partial(_scatter_body, ept, nch),
        out_type=[
            jax.ShapeDtypeStruct((NC, N_PAD, D), jnp.float32),
            jax.ShapeDtypeStruct((NC, N_PAD, 16), jnp.float32),
        ],
        mesh=mesh,
        scratch_types=[
            pltpu.VMEM((nch, CH2), jnp.int32),
            pltpu.VMEM((nch, CH2), jnp.int32),
            pltpu.VMEM((CH2, D), jnp.float32),
            pltpu.VMEM((CH2, D), jnp.float32),
            pltpu.VMEM((CH2, D), jnp.float32),
            pltpu.VMEM((CH2, D), jnp.float32),
            pltpu.VMEM((CH2, 16), jnp.float32),
            pltpu.VMEM_SHARED((N_PAD, D), jnp.float32),
            pltpu.VMEM_SHARED((N_PAD, 16), jnp.float32),
            pltpu.SemaphoreType.DMA,
            pltpu.SemaphoreType.DMA,
            pltpu.SemaphoreType.DMA,
            pltpu.SemaphoreType.DMA,
        ],
        compiler_params=pltpu.CompilerParams(use_tc_tiling_on_sc=False),
    )
    return f(ir3_e, ir3_o, he, ho)


BN = 1280


def _final_body(*refs):
    parts = refs[:NB]
    cnts = refs[NB:2 * NB]
    w3_ref, b3_ref, out_ref = refs[2 * NB:]
    p = parts[0][0] + parts[0][1]
    cnt = cnts[0][0, :, 0:1] + cnts[0][1, :, 0:1]
    for i in range(1, NB):
        p = p + parts[i][0] + parts[i][1]
        cnt = cnt + cnts[i][0, :, 0:1] + cnts[i][1, :, 0:1]
    out_ref[...] = (
        jnp.dot(p, w3_ref[...], preferred_element_type=jnp.float32)
        + cnt * b3_ref[...]
    )


def _final(parts, cnts, w3, b3):
    return pl.pallas_call(
        _final_body,
        grid=(N_PAD // BN,),
        in_specs=(
            [pl.BlockSpec((NC, BN, D), lambda i: (0, i, 0))] * NB
            + [pl.BlockSpec((NC, BN, 16), lambda i: (0, i, 0))] * NB
            + [
                pl.BlockSpec((D, D), lambda i: (0, 0)),
                pl.BlockSpec((1, D), lambda i: (0, 0)),
            ]
        ),
        out_specs=pl.BlockSpec((BN, D), lambda i: (i, 0)),
        out_shape=jax.ShapeDtypeStruct((N_PAD, D), jnp.float32),
    )(*parts, *cnts, w3, b3)


def kernel(x, edge_idx, W1, b1, W2, b2, W3, b3):
    row = edge_idx[0].astype(jnp.int32)
    col = edge_idx[1].astype(jnp.int32)
    xa, xb = _precompute(x, W1[:D], W1[D:], b1.reshape(1, D))

    ept = EB // NW
    nch = ept // CH
    gs, rows = [], []
    for k in range(NB):
        sl = slice(k * EB, (k + 1) * EB)
        rk, ck = row[sl], col[sl]
        r3 = rk.reshape(NW, nch, CH)
        c3 = ck.reshape(NW, nch, CH)
        gs.append(_gather(r3, c3, xa, xb, EB))
        rows.append(rk)
    parts, cnts = [], []
    for k in range(NB):
        he, ho = _mlp(gs[k], W2, b2.reshape(1, D))
        r3e = rows[k][0::2].reshape(NW, nch, CH2)
        r3o = rows[k][1::2].reshape(NW, nch, CH2)
        p, ct = _scatter(r3e, r3o, he, ho)
        parts.append(p)
        cnts.append(ct)

    return _final(parts, cnts, W3, b3.reshape(1, D))[:N_NODES]


# speedup vs baseline: 1.0010x; 1.0010x over previous
import functools

import jax
import jax.numpy as jnp
from jax import lax
from jax.experimental import pallas as pl
from jax.experimental.pallas import tpu as pltpu
from jax.experimental.pallas import tpu_sc as plsc

N_NODES = 10000
N_PAD = 10240
N_EDGES = 320000
D = 128
HD = D // 2

NC = 2
NS = 16
NW = NC * NS
CH = 80
CH2 = CH // 2
ROWS_PER_TILE = N_PAD // NS
ZR = 40

NB = 5
EB = N_EDGES // NB


def _pipelined_chunks(nch, start, wait0, wait1, finish):
    start(0, 0)
    npair = (nch - 1) // 2 if nch % 2 else nch // 2 - 1

    def pair(jj, carry):
        j0 = 2 * jj
        start(j0 + 1, 1)
        wait0()
        finish(j0, 0)
        start(j0 + 2, 0)
        wait1()
        finish(j0 + 1, 1)
        return carry

    lax.fori_loop(0, npair, pair, 0)
    if nch % 2:
        wait0()
        finish(nch - 1, 0)
    else:
        j = 2 * npair
        start(j + 1, 1)
        wait0()
        finish(j, 0)
        wait1()
        finish(j + 1, 1)


def _pack_i32(v32):
    b = jax.lax.bitcast_convert_type(v32, jnp.int32) + jnp.int32(32768)
    lo = jax.lax.shift_right_logical(b[:, :HD], 16)
    hi = b[:, HD:] & jnp.int32(-65536)
    return lo | hi


def _unpack_f32(w):
    lo = jax.lax.bitcast_convert_type(jax.lax.shift_left(w, 16), jnp.float32)
    hi = jax.lax.bitcast_convert_type(w & jnp.int32(-65536), jnp.float32)
    return lo, hi


def _pre_body(x_ref, w1a_ref, w1b_ref, b1_ref, xa_ref, xb_ref):
    x = x_ref[...]
    xa_ref[...] = _pack_i32(
        jnp.dot(x, w1a_ref[...], preferred_element_type=jnp.float32)
        + b1_ref[...]
    )
    xb_ref[...] = _pack_i32(
        jnp.dot(x, w1b_ref[...], preferred_element_type=jnp.float32)
    )


def _precompute(x, w1a, w1b, b1):
    return pl.pallas_call(
        _pre_body,
        out_shape=[
            jax.ShapeDtypeStruct((N_NODES, HD), jnp.int32),
            jax.ShapeDtypeStruct((N_NODES, HD), jnp.int32),
        ],
    )(x, w1a, w1b, b1)


def _gather_body(ept, nch, row3d, col3d, xa, xb, g,
                 ir, ic, ba0, bb0, ba1, bb1, go0, go1,
                 sa0, sb0, sa1, sb1):
    wid = lax.axis_index("s") * NC + lax.axis_index("c")
    base = wid * ept
    pltpu.sync_copy(row3d.at[wid], ir)
    pltpu.sync_copy(col3d.at[wid], ic)
    bufs = ((ba0, bb0, go0, sa0, sb0), (ba1, bb1, go1, sa1, sb1))

    def start(j, slot):
        ba, bb, _, sa, sb = bufs[slot]
        pltpu.async_copy(xa.at[ir.at[j]], ba, sa)
        pltpu.async_copy(xb.at[ic.at[j]], bb, sb)

    def wait(slot):
        ba, bb, _, sa, sb = bufs[slot]
        pltpu.make_async_copy(xa.at[ir.at[0]], ba, sa).wait()
        pltpu.make_async_copy(xb.at[ic.at[0]], bb, sb).wait()

    def finish(j, slot):
        ba, bb, go = bufs[slot][:3]

        def addrow(r2, carry):
            for half in range(2):
                r = 2 * r2 + half
                for k in range(HD // 16):
                    sl = pl.ds(k * 16, 16)
                    a = plsc.bitcast(ba[r, sl], jnp.bfloat16)
                    b = plsc.bitcast(bb[r, sl], jnp.bfloat16)
                    go[r2, pl.ds(half * HD + k * 16, 16)] = plsc.bitcast(
                        a + b, jnp.int32
                    )
            return carry

        lax.fori_loop(0, CH2, addrow, 0)
        pltpu.sync_copy(go, g.at[pl.ds((base + j * CH) // 2, CH2)])

    _pipelined_chunks(nch, start, lambda: wait(0), lambda: wait(1), finish)


def _gather(row3d, col3d, xa, xb, n_edges):
    ept = n_edges // NW
    nch = ept // CH
    mesh = plsc.VectorSubcoreMesh(core_axis_name="c", subcore_axis_name="s")
    f = pl.kernel(
        functools.partial(_gather_body, ept, nch),
        out_type=jax.ShapeDtypeStruct((n_edges // 2, D), jnp.int32),
        mesh=mesh,
        scratch_types=[
            pltpu.VMEM((nch, CH), jnp.int32),
            pltpu.VMEM((nch, CH), jnp.int32),
            pltpu.VMEM((CH, HD), jnp.int32),
            pltpu.VMEM((CH, HD), jnp.int32),
            pltpu.VMEM((CH, HD), jnp.int32),
            pltpu.VMEM((CH, HD), jnp.int32),
            pltpu.VMEM((CH2, D), jnp.int32),
            pltpu.VMEM((CH2, D), jnp.int32),
            pltpu.SemaphoreType.DMA,
            pltpu.SemaphoreType.DMA,
            pltpu.SemaphoreType.DMA,
            pltpu.SemaphoreType.DMA,
        ],
        compiler_params=pltpu.CompilerParams(
            use_tc_tiling_on_sc=False, needs_layout_passes=False
        ),
    )
    return f(row3d, col3d, xa, xb)


BE = 1280
BE2 = BE // 2


def _mlp_body(g_ref, w2_ref, b2_ref, he_ref, ho_ref):
    lo, hi = _unpack_f32(g_ref[...])
    ge = jnp.concatenate([lo[:, :HD], hi[:, :HD]], axis=1)
    go = jnp.concatenate([lo[:, HD:], hi[:, HD:]], axis=1)
    w2 = w2_ref[...]
    b2 = b2_ref[...]
    for gg, out_ref in ((ge, he_ref), (go, ho_ref)):
        a = jax.nn.softplus(gg)
        z = jnp.dot(a, w2, preferred_element_type=jnp.float32) + b2
        out_ref[...] = jax.nn.softplus(z)


def _mlp(g, w2, b2):
    n2 = g.shape[0]
    return pl.pallas_call(
        _mlp_body,
        grid=(n2 // BE2,),
        in_specs=[
            pl.BlockSpec((BE2, D), lambda i: (i, 0)),
            pl.BlockSpec((D, D), lambda i: (0, 0)),
            pl.BlockSpec((1, D), lambda i: (0, 0)),
        ],
        out_specs=[
            pl.BlockSpec((BE2, D), lambda i: (i, 0)),
            pl.BlockSpec((BE2, D), lambda i: (i, 0)),
        ],
        out_shape=[
            jax.ShapeDtypeStruct((n2, D), jnp.float32),
            jax.ShapeDtypeStruct((n2, D), jnp.float32),
        ],
    )(g, w2, b2)


def _scatter_body(ept, nch, ir3_e, ir3_o, he, ho, parts, cnts,
                  ire, iro, he0, ho0, he1, ho1, ones, acc, acc_c,
                  se0, so0, se1, so1):
    c = lax.axis_index("c")
    s = lax.axis_index("s")
    wid = s * NC + c
    base2 = wid * ept // 2

    zv = jnp.zeros((16,), jnp.float32)
    ov = jnp.ones((16,), jnp.float32)

    def fill_z(r, carry):
        for k in range(D // 16):
            he0[r, pl.ds(k * 16, 16)] = zv
        return carry

    lax.fori_loop(0, CH2, fill_z, 0)

    def fill_zc(r, carry):
        ones[r, :] = zv
        return carry

    lax.fori_loop(0, CH2, fill_zc, 0)

    def zero_stripe(t, carry):
        off = s * ROWS_PER_TILE + t * ZR
        pltpu.sync_copy(he0.at[pl.ds(0, ZR)], acc.at[pl.ds(off, ZR)])
        pltpu.sync_copy(ones.at[pl.ds(0, ZR)], acc_c.at[pl.ds(off, ZR)])
        return carry

    lax.fori_loop(0, ROWS_PER_TILE // ZR, zero_stripe, 0)

    def fill_ones(r, carry):
        ones[r, :] = ov
        return carry

    lax.fori_loop(0, CH2, fill_ones, 0)
    pltpu.sync_copy(ir3_e.at[wid], ire)
    pltpu.sync_copy(ir3_o.at[wid], iro)
    plsc.subcore_barrier()

    bufs = ((he0, ho0, se0, so0), (he1, ho1, se1, so1))

    def start(j, slot):
        hbe, hbo, se, so = bufs[slot]
        pltpu.async_copy(he.at[pl.ds(base2 + j * CH2, CH2)], hbe, se)
        pltpu.async_copy(ho.at[pl.ds(base2 + j * CH2, CH2)], hbo, so)

    def wait(slot):
        hbe, hbo, se, so = bufs[slot]
        pltpu.make_async_copy(he.at[pl.ds(0, CH2)], hbe, se).wait()
        pltpu.make_async_copy(ho.at[pl.ds(0, CH2)], hbo, so).wait()

    def finish(j, slot):
        hbe, hbo = bufs[slot][:2]
        pltpu.sync_copy(hbe, acc.at[ire.at[j]], add=True)
        pltpu.sync_copy(hbo, acc.at[iro.at[j]], add=True)
        pltpu.sync_copy(ones, acc_c.at[ire.at[j]], add=True)
        pltpu.sync_copy(ones, acc_c.at[iro.at[j]], add=True)

    _pipelined_chunks(nch, start, lambda: wait(0), lambda: wait(1), finish)
    plsc.subcore_barrier()

    stripe = pl.ds(s * ROWS_PER_TILE, ROWS_PER_TILE)
    pltpu.sync_copy(acc.at[stripe], parts.at[c].at[stripe])
    pltpu.sync_copy(acc_c.at[stripe], cnts.at[c].at[stripe])


def _scatter(ir3_e, ir3_o, he, ho):
    n_edges = 2 * he.shape[0]
    ept = n_edges // NW
    nch = ept // CH
    mesh = plsc.VectorSubcoreMesh(core_axis_name="c", subcore_axis_name="s")
    f = pl.kernel(
        functools.partial(_scatter_body, ept, nch),
        out_type=[
            jax.ShapeDtypeStruct((NC, N_PAD, D), jnp.float32),
            jax.ShapeDtypeStruct((NC, N_PAD, 16), jnp.float32),
        ],
        mesh=mesh,
        scratch_types=[
            pltpu.VMEM((nch, CH2), jnp.int32),
            pltpu.VMEM((nch, CH2), jnp.int32),
            pltpu.VMEM((CH2, D), jnp.float32),
            pltpu.VMEM((CH2, D), jnp.float32),
            pltpu.VMEM((CH2, D), jnp.float32),
            pltpu.VMEM((CH2, D), jnp.float32),
            pltpu.VMEM((CH2, 16), jnp.float32),
            pltpu.VMEM_SHARED((N_PAD, D), jnp.float32),
            pltpu.VMEM_SHARED((N_PAD, 16), jnp.float32),
            pltpu.SemaphoreType.DMA,
            pltpu.SemaphoreType.DMA,
            pltpu.SemaphoreType.DMA,
            pltpu.SemaphoreType.DMA,
        ],
        compiler_params=pltpu.CompilerParams(use_tc_tiling_on_sc=False),
    )
    return f(ir3_e, ir3_o, he, ho)


BN = 1280


def _final_body(*refs):
    parts = refs[:NB]
    cnts = refs[NB:2 * NB]
    w3_ref, b3_ref, out_ref = refs[2 * NB:]
    p = parts[0][0] + parts[0][1]
    cnt = cnts[0][0, :, 0:1] + cnts[0][1, :, 0:1]
    for i in range(1, NB):
        p = p + parts[i][0] + parts[i][1]
        cnt = cnt + cnts[i][0, :, 0:1] + cnts[i][1, :, 0:1]
    out_ref[...] = (
        jnp.dot(p, w3_ref[...], preferred_element_type=jnp.float32)
        + cnt * b3_ref[...]
    )


def _final(parts, cnts, w3, b3):
    return pl.pallas_call(
        _final_body,
        grid=(N_PAD // BN,),
        in_specs=(
            [pl.BlockSpec((NC, BN, D), lambda i: (0, i, 0))] * NB
            + [pl.BlockSpec((NC, BN, 16), lambda i: (0, i, 0))] * NB
            + [
                pl.BlockSpec((D, D), lambda i: (0, 0)),
                pl.BlockSpec((1, D), lambda i: (0, 0)),
            ]
        ),
        out_specs=pl.BlockSpec((BN, D), lambda i: (i, 0)),
        out_shape=jax.ShapeDtypeStruct((N_PAD, D), jnp.float32),
    )(*parts, *cnts, w3, b3)


def kernel(x, edge_idx, W1, b1, W2, b2, W3, b3):
    row = edge_idx[0].astype(jnp.int32)
    col = edge_idx[1].astype(jnp.int32)
    xa, xb = _precompute(x, W1[:D], W1[D:], b1.reshape(1, D))

    ept = EB // NW
    nch = ept // CH
    gs, rows = [], []
    for k in range(NB):
        sl = slice(k * EB, (k + 1) * EB)
        rk, ck = row[sl], col[sl]
        r3 = rk.reshape(NW, nch, CH)
        c3 = ck.reshape(NW, nch, CH)
        gs.append(_gather(r3, c3, xa, xb, EB))
        rows.append(rk)
    parts, cnts = [], []
    for k in range(NB):
        he, ho = _mlp(gs[k], W2, b2.reshape(1, D))
        r3e = rows[k][0::2].reshape(NW, nch, CH2)
        r3o = rows[k][1::2].reshape(NW, nch, CH2)
        p, ct = _scatter(r3e, r3o, he, ho)
        parts.append(p)
        cnts.append(ct)

    return _final(parts, cnts, W3, b3.reshape(1, D))[:N_NODES]
